# baseline (speedup 1.0000x reference)
import functools

import jax
import jax.numpy as jnp
from jax import lax
from jax.experimental import pallas as pl
from jax.experimental.pallas import tpu as pltpu
from jax.experimental.pallas import tpu_sc as plsc

_NC = 2
_NS = 16
_L = 16
_NW = _NC * _NS


def kernel(opinions, weights):
    E, B, d = opinions.shape
    b_per_w = B // _NW
    CH = 64
    n_ch = b_per_w // CH
    n_grp = b_per_w // _L
    d_vecs = d // _L

    mesh = plsc.VectorSubcoreMesh(core_axis_name="c", subcore_axis_name="s")

    @functools.partial(
        pl.kernel,
        out_type=jax.ShapeDtypeStruct((B * d,), jnp.float32),
        mesh=mesh,
        scratch_types=[
            pltpu.VMEM((b_per_w * E,), jnp.float32),
            pltpu.VMEM((b_per_w,), jnp.int32),
            pltpu.VMEM((E * d,), jnp.float32),
            pltpu.VMEM((CH * d,), jnp.float32),
            pltpu.VMEM((CH * d,), jnp.float32),
            pltpu.SemaphoreType.DMA,
        ],
        compiler_params=pltpu.CompilerParams(needs_layout_passes=False),
    )
    def k(op_hbm, w_hbm, out_hbm, w_v, idx_v, table_v, rows_a, rows_b, wsem):
        wid = lax.axis_index("s") * _NC + lax.axis_index("c")
        base = wid * b_per_w

        pltpu.sync_copy(w_hbm.at[pl.ds(base * E, b_per_w * E)], w_v)
        pltpu.sync_copy(op_hbm.at[pl.ds(0, E * d)], table_v)

        iota = lax.iota(jnp.int32, _L)

        def argmax_group(g, _):
            fvec = (g * _L + iota) * E
            best_v = plsc.load_gather(w_v, [fvec])
            best_i = jnp.zeros((_L,), jnp.int32)
            for e in range(1, E):
                v = plsc.load_gather(w_v, [fvec + e])
                p = v > best_v
                best_v = jnp.where(p, v, best_v)
                best_i = jnp.where(p, e * d, best_i)
            idx_v[pl.ds(g * _L, _L)] = best_i
            return 0

        lax.fori_loop(0, n_grp, argmax_group, 0)

        bufs = [rows_a, rows_b]

        def materialize(c, buf):
            def one_row(r, _):
                src = plsc.load_gather(
                    idx_v, [jnp.zeros((_L,), jnp.int32) + (c * CH + r)])
                src = src + iota
                dst = r * d
                for j in range(d_vecs):
                    buf[pl.ds(dst + j * _L, _L)] = plsc.load_gather(
                        table_v, [src + j * _L])
                return 0

            lax.fori_loop(0, CH, one_row, 0)

        writes = [None, None]
        for c in range(n_ch):
            b = c & 1
            if writes[b] is not None:
                writes[b].wait()
            materialize(c, bufs[b])
            writes[b] = pltpu.async_copy(
                bufs[b],
                out_hbm.at[pl.ds((base + c * CH) * d, CH * d)], wsem)
        for w in writes:
            if w is not None:
                w.wait()

    out = k(opinions.reshape(E * B * d), weights.reshape(B * E))
    return out.reshape(B, d)


# speedup vs baseline: 1.1341x; 1.1341x over previous
import functools

import jax
import jax.numpy as jnp
from jax import lax
from jax.experimental import pallas as pl
from jax.experimental.pallas import tpu as pltpu
from jax.experimental.pallas import tpu_sc as plsc

_NC = 2
_NS = 16
_L = 16
_NW = _NC * _NS


def kernel(opinions, weights):
    E, B, d = opinions.shape
    b_per_w = B // _NW
    CH = 64
    n_ch = b_per_w // CH
    n_grp = b_per_w // _L
    d_vecs = d // _L

    mesh = plsc.VectorSubcoreMesh(core_axis_name="c", subcore_axis_name="s")

    @functools.partial(
        pl.kernel,
        out_type=jax.ShapeDtypeStruct((B * d,), jnp.float32),
        mesh=mesh,
        scratch_types=[
            pltpu.VMEM((b_per_w * E,), jnp.float32),
            pltpu.VMEM((b_per_w,), jnp.int32),
            pltpu.VMEM((E * d,), jnp.float32),
            pltpu.SemaphoreType.DMA,
        ],
        compiler_params=pltpu.CompilerParams(needs_layout_passes=False),
    )
    def k(op_hbm, w_hbm, out_hbm, w_v, idx_v, table_v, wsem):
        wid = lax.axis_index("s") * _NC + lax.axis_index("c")
        base = wid * b_per_w

        pltpu.sync_copy(w_hbm.at[pl.ds(base * E, b_per_w * E)], w_v)
        pltpu.sync_copy(op_hbm.at[pl.ds(0, E * d)], table_v)

        iota = lax.iota(jnp.int32, _L)

        def argmax_group(g, _):
            fvec = (g * _L + iota) * E
            best_v = plsc.load_gather(w_v, [fvec])
            best_i = jnp.zeros((_L,), jnp.int32)
            for e in range(1, E):
                v = plsc.load_gather(w_v, [fvec + e])
                p = v > best_v
                best_v = jnp.where(p, v, best_v)
                best_i = jnp.where(p, e * d, best_i)
            idx_v[pl.ds(g * _L, _L)] = best_i
            return 0

        lax.fori_loop(0, n_grp, argmax_group, 0)

        writes = []
        for g in range(n_grp):
            ev = idx_v[pl.ds(g * _L, _L)]
            for u in range(_L):
                r = g * _L + u
                writes.append(pltpu.async_copy(
                    table_v.at[pl.ds(pl.multiple_of(ev[u], 256), d)],
                    out_hbm.at[pl.ds((base + r) * d, d)], wsem))
        for w in writes:
            w.wait()

    out = k(opinions.reshape(E * B * d), weights.reshape(B * E))
    return out.reshape(B, d)
